# Initial kernel scaffold; baseline (speedup 1.0000x reference)
import jax
import jax.numpy as jnp
from jax.experimental import pallas as pl


def kernel(adj, aug_feat1, aug_feat2, g1_W1, g1_as1, g1_ad1, g1_b1, g1_W2, g1_as2, g1_ad2, g1_b2, g2_W1, g2_as1, g2_ad1, g2_b1, g2_W2, g2_as2, g2_ad2, g2_b2, c_W1, c_b1, c_W2, c_b2, a_W1, a_b1, a_W2):
    raise NotImplementedError("write your pallas kernel here")


# speedup vs baseline: 12.6201x; 12.6201x over previous
import functools

import jax
import jax.numpy as jnp
from jax import lax
from jax.experimental import pallas as pl
from jax.experimental.pallas import tpu as pltpu
from jax.experimental.pallas import tpu_sc as plsc

N = 10000
D = 128
NP = 10240
NC = 2
NS = 16
NW = NC * NS
K = 128
E = 320000
CPW = -(-E // (NW * K))
EPAD = NW * CPW * K
RPT = NP // NS

def _zero_vmem(rows_v, den_v):
    z16 = jnp.zeros((16,), jnp.float32)

    def zrow(i, _):
        for j in range(D // 16):
            rows_v[i, pl.ds(j * 16, 16)] = z16
        return 0
    lax.fori_loop(0, K, zrow, 0)

    def zden(i, _):
        den_v[pl.ds(pl.multiple_of(i * 16, 16), 16)] = z16
        return 0
    lax.fori_loop(0, NP // 16, zden, 0)


def _zero_num_shared(rows_v, num_sh, sid):
    for r in range(RPT // K):
        pltpu.sync_copy(rows_v, num_sh.at[pl.ds(sid * RPT + r * K, K)])


def _scale_rows(rows_v, w_v):
    def scale(g, _):
        w16 = w_v[pl.ds(pl.multiple_of(g * 16, 16), 16)]
        for l in range(16):
            ws = w16[l]
            row = g * 16 + l
            for j in range(D // 16):
                sl = pl.ds(j * 16, 16)
                rows_v[row, sl] = rows_v[row, sl] * ws
        return 0
    lax.fori_loop(0, K // 16, scale, 0)


def _edge_pass(src_h, dst_h, h_h, num_h, num_sh, den_v, src_v, dst_v, w_v,
               rows_v, sem, weight_fn, cid, sid):
    gw = cid * NS + sid

    def chunk(c, _):
        base = (gw * CPW + c) * K
        pltpu.sync_copy(src_h.at[pl.ds(base, K)], src_v)
        pltpu.sync_copy(dst_h.at[pl.ds(base, K)], dst_v)
        cp = pltpu.async_copy(h_h.at[src_v], rows_v, sem)
        for g in range(K // 16):
            s16 = src_v[pl.ds(g * 16, 16)]
            d16 = dst_v[pl.ds(g * 16, 16)]
            w = weight_fn(s16, d16)
            w_v[pl.ds(g * 16, 16)] = w
            plsc.addupdate_scatter(den_v, [d16], w)
        cp.wait()
        _scale_rows(rows_v, w_v)
        pltpu.sync_copy(rows_v, num_sh.at[dst_v], add=True)
        return 0
    lax.fori_loop(0, CPW, chunk, 0)

    plsc.subcore_barrier()
    pltpu.sync_copy(num_sh.at[pl.ds(sid * RPT, RPT)],
                    num_h.at[cid, pl.ds(sid * RPT, RPT)])


def _gat_edges_body(src_h, dst_h, h_h, asrc_h, adst_h, num_h, den_h,
                    num_sh, asrc_v, adst_v, den_v, src_v, dst_v, w_v, rows_v,
                    sem):
    cid = lax.axis_index("c")
    sid = lax.axis_index("s")
    _zero_vmem(rows_v, den_v)
    _zero_num_shared(rows_v, num_sh, sid)
    pltpu.sync_copy(asrc_h, asrc_v)
    pltpu.sync_copy(adst_h, adst_v)
    plsc.subcore_barrier()

    def weight(s16, d16):
        e = plsc.load_gather(asrc_v, [s16]) + plsc.load_gather(adst_v, [d16])
        e = jnp.where(e >= 0.0, e, 0.2 * e)
        return jnp.exp(e)

    _edge_pass(src_h, dst_h, h_h, num_h, num_sh, den_v, src_v, dst_v, w_v,
               rows_v, sem, weight, cid, sid)
    pltpu.sync_copy(den_v, den_h.at[cid * NS + sid])


def _gcn_edges_body(src_h, dst_h, h_h, dinv_h, num_h,
                    num_sh, dinv_v, den_v, src_v, dst_v, w_v, rows_v, sem):
    cid = lax.axis_index("c")
    sid = lax.axis_index("s")
    _zero_vmem(rows_v, den_v)
    _zero_num_shared(rows_v, num_sh, sid)
    pltpu.sync_copy(dinv_h, dinv_v)
    plsc.subcore_barrier()

    def weight(s16, d16):
        return plsc.load_gather(dinv_v, [s16]) * plsc.load_gather(dinv_v, [d16])

    _edge_pass(src_h, dst_h, h_h, num_h, num_sh, den_v, src_v, dst_v, w_v,
               rows_v, sem, weight, cid, sid)


def _degrees_body(dst_h, deg_h, den_v, dst_v):
    cid = lax.axis_index("c")
    sid = lax.axis_index("s")
    gw = cid * NS + sid

    z16 = jnp.zeros((16,), jnp.float32)

    def zden(i, _):
        den_v[pl.ds(pl.multiple_of(i * 16, 16), 16)] = z16
        return 0
    lax.fori_loop(0, NP // 16, zden, 0)

    ones = jnp.ones((16,), jnp.float32)

    def chunk(c, _):
        base = (gw * CPW + c) * K
        pltpu.sync_copy(dst_h.at[pl.ds(base, K)], dst_v)
        for g in range(K // 16):
            d16 = dst_v[pl.ds(g * 16, 16)]
            plsc.addupdate_scatter(den_v, [d16], ones)
        return 0
    lax.fori_loop(0, CPW, chunk, 0)
    pltpu.sync_copy(den_v, deg_h.at[gw])


@functools.lru_cache(maxsize=None)
def _sc_kernels():
    mesh = plsc.VectorSubcoreMesh(core_axis_name="c", subcore_axis_name="s",
                                  num_cores=NC, num_subcores=NS)
    cparams = pltpu.CompilerParams(needs_layout_passes=False)
    gat = pl.kernel(
        _gat_edges_body,
        out_type=(jax.ShapeDtypeStruct((NC, NP, D), jnp.float32),
                  jax.ShapeDtypeStruct((NW, NP), jnp.float32)),
        mesh=mesh,
        compiler_params=cparams,
        scratch_types=[
            pltpu.VMEM_SHARED((NP, D), jnp.float32),
            pltpu.VMEM((NP,), jnp.float32),
            pltpu.VMEM((NP,), jnp.float32),
            pltpu.VMEM((NP,), jnp.float32),
            pltpu.VMEM((K,), jnp.int32),
            pltpu.VMEM((K,), jnp.int32),
            pltpu.VMEM((K,), jnp.float32),
            pltpu.VMEM((K, D), jnp.float32),
            pltpu.SemaphoreType.DMA,
        ],
    )
    gcn = pl.kernel(
        _gcn_edges_body,
        out_type=jax.ShapeDtypeStruct((NC, NP, D), jnp.float32),
        mesh=mesh,
        compiler_params=cparams,
        scratch_types=[
            pltpu.VMEM_SHARED((NP, D), jnp.float32),
            pltpu.VMEM((NP,), jnp.float32),
            pltpu.VMEM((NP,), jnp.float32),
            pltpu.VMEM((K,), jnp.int32),
            pltpu.VMEM((K,), jnp.int32),
            pltpu.VMEM((K,), jnp.float32),
            pltpu.VMEM((K, D), jnp.float32),
            pltpu.SemaphoreType.DMA,
        ],
    )
    deg = pl.kernel(
        _degrees_body,
        out_type=jax.ShapeDtypeStruct((NW, NP), jnp.float32),
        mesh=mesh,
        compiler_params=cparams,
        scratch_types=[
            pltpu.VMEM((NP,), jnp.float32),
            pltpu.VMEM((K,), jnp.int32),
        ],
    )
    return gat, gcn, deg


_BM = 256


def _mm(x, W):
    def body(x_ref, w_ref, o_ref):
        o_ref[...] = jnp.dot(x_ref[...], w_ref[...],
                             preferred_element_type=jnp.float32)
    return pl.pallas_call(
        body,
        grid=(NP // _BM,),
        in_specs=[pl.BlockSpec((_BM, D), lambda i: (i, 0)),
                  pl.BlockSpec((D, D), lambda i: (0, 0))],
        out_specs=pl.BlockSpec((_BM, D), lambda i: (i, 0)),
        out_shape=jax.ShapeDtypeStruct((NP, D), jnp.float32),
    )(x, W)


def _mm_att(x, W, a2):
    def body(x_ref, w_ref, a_ref, h_ref, as_ref, ad_ref):
        h = jnp.dot(x_ref[...], w_ref[...], preferred_element_type=jnp.float32)
        av = jnp.dot(h, a_ref[...], preferred_element_type=jnp.float32)
        h_ref[...] = h
        as_ref[...] = av[:, 0:1]
        ad_ref[...] = av[:, 1:2]
    return pl.pallas_call(
        body,
        grid=(NP // _BM,),
        in_specs=[pl.BlockSpec((_BM, D), lambda i: (i, 0)),
                  pl.BlockSpec((D, D), lambda i: (0, 0)),
                  pl.BlockSpec((D, 2), lambda i: (0, 0))],
        out_specs=[pl.BlockSpec((_BM, D), lambda i: (i, 0)),
                   pl.BlockSpec((_BM, 1), lambda i: (i, 0)),
                   pl.BlockSpec((_BM, 1), lambda i: (i, 0))],
        out_shape=[jax.ShapeDtypeStruct((NP, D), jnp.float32),
                   jax.ShapeDtypeStruct((NP, 1), jnp.float32),
                   jax.ShapeDtypeStruct((NP, 1), jnp.float32)],
    )(x, W, a2)


def _gat_epilogue(num, den, h, asrc, adst, b, relu):
    def body(n_ref, d_ref, h_ref, as_ref, ad_ref, b_ref, o_ref):
        e = as_ref[...] + ad_ref[...]
        w_self = jnp.exp(jnp.where(e >= 0.0, e, 0.2 * e))
        den_tot = jnp.sum(d_ref[...], axis=0, keepdims=True).T + w_self + 1e-16
        numer = n_ref[0] + n_ref[1] + w_self * h_ref[...]
        out = numer / den_tot + b_ref[...]
        if relu:
            out = jnp.maximum(out, 0.0)
        o_ref[...] = out
    return pl.pallas_call(
        body,
        grid=(NP // _BM,),
        in_specs=[pl.BlockSpec((NC, _BM, D), lambda i: (0, i, 0)),
                  pl.BlockSpec((NW, _BM), lambda i: (0, i)),
                  pl.BlockSpec((_BM, D), lambda i: (i, 0)),
                  pl.BlockSpec((_BM, 1), lambda i: (i, 0)),
                  pl.BlockSpec((_BM, 1), lambda i: (i, 0)),
                  pl.BlockSpec((1, D), lambda i: (0, 0))],
        out_specs=pl.BlockSpec((_BM, D), lambda i: (i, 0)),
        out_shape=jax.ShapeDtypeStruct((NP, D), jnp.float32),
    )(num, den, h, asrc, adst, b.reshape(1, D))


def _gcn_epilogue(num, h, dinv, b, relu):
    def body(n_ref, h_ref, di_ref, b_ref, o_ref):
        di = di_ref[...]
        out = n_ref[0] + n_ref[1] + (di * di) * h_ref[...] + b_ref[...]
        if relu:
            out = jnp.maximum(out, 0.0)
        o_ref[...] = out
    return pl.pallas_call(
        body,
        grid=(NP // _BM,),
        in_specs=[pl.BlockSpec((NC, _BM, D), lambda i: (0, i, 0)),
                  pl.BlockSpec((_BM, D), lambda i: (i, 0)),
                  pl.BlockSpec((_BM, 1), lambda i: (i, 0)),
                  pl.BlockSpec((1, D), lambda i: (0, 0))],
        out_specs=pl.BlockSpec((_BM, D), lambda i: (i, 0)),
        out_shape=jax.ShapeDtypeStruct((NP, D), jnp.float32),
    )(num, h, dinv, b.reshape(1, D))


def _dinv_kernel(deg):
    def body(deg_ref, o_ref):
        tot = jnp.sum(deg_ref[...], axis=0, keepdims=True) + 1.0
        o_ref[...] = lax.rsqrt(tot)
    return pl.pallas_call(
        body,
        out_shape=jax.ShapeDtypeStruct((1, NP), jnp.float32),
    )(deg)


def _fusion(e1, e2, c1, c2, aW1, ab1, aW2):
    def body(e1_ref, e2_ref, c1_ref, c2_ref, w1_ref, b1_ref, w2_ref, o_ref):
        z0 = e1_ref[...]
        z1 = e2_ref[...]
        z2 = (c1_ref[...] + c2_ref[...]) * 0.5

        def att(z):
            t = jnp.tanh(jnp.dot(z, w1_ref[...],
                                 preferred_element_type=jnp.float32)
                         + b1_ref[...])
            return jnp.dot(t, w2_ref[...], preferred_element_type=jnp.float32)

        w0, w1, w2 = att(z0), att(z1), att(z2)
        m = jnp.maximum(jnp.maximum(w0, w1), w2)
        x0 = jnp.exp(w0 - m)
        x1 = jnp.exp(w1 - m)
        x2 = jnp.exp(w2 - m)
        s = x0 + x1 + x2
        o_ref[...] = (x0 * z0 + x1 * z1 + x2 * z2) / s
    return pl.pallas_call(
        body,
        grid=(NP // _BM,),
        in_specs=[pl.BlockSpec((_BM, D), lambda i: (i, 0))] * 4
                 + [pl.BlockSpec((D, 16), lambda i: (0, 0)),
                    pl.BlockSpec((1, 16), lambda i: (0, 0)),
                    pl.BlockSpec((16, 1), lambda i: (0, 0))],
        out_specs=pl.BlockSpec((_BM, D), lambda i: (i, 0)),
        out_shape=jax.ShapeDtypeStruct((NP, D), jnp.float32),
    )(e1, e2, c1, c2, aW1, ab1, aW2)


def _gat_layer(x, W, a_s, a_d, b, src, dst, relu):
    gat_edges, _, _ = _sc_kernels()
    h, asrc, adst = _mm_att(x, W, jnp.stack([a_s, a_d], axis=1))
    num, den = gat_edges(src, dst, h, asrc.reshape(NP), adst.reshape(NP))
    return _gat_epilogue(num, den, h, asrc, adst, b, relu)


def _gcn_layer(x, W, b, src, dst, dinv_flat, dinv_col, relu):
    _, gcn_edges, _ = _sc_kernels()
    h = _mm(x, W)
    num = gcn_edges(src, dst, h, dinv_flat)
    return _gcn_epilogue(num, h, dinv_col, b, relu)


def kernel(adj, aug_feat1, aug_feat2,
           g1_W1, g1_as1, g1_ad1, g1_b1, g1_W2, g1_as2, g1_ad2, g1_b2,
           g2_W1, g2_as1, g2_ad1, g2_b1, g2_W2, g2_as2, g2_ad2, g2_b2,
           c_W1, c_b1, c_W2, c_b2, a_W1, a_b1, a_W2):
    pad_e = jnp.full((EPAD - E,), N, jnp.int32)
    src = jnp.concatenate([adj[0], pad_e])
    dst = jnp.concatenate([adj[1], pad_e])
    x1 = jnp.pad(aug_feat1, ((0, NP - N), (0, 0)))
    x2 = jnp.pad(aug_feat2, ((0, NP - N), (0, 0)))

    _, _, degrees = _sc_kernels()
    deg = degrees(dst)
    dinv = _dinv_kernel(deg)
    dinv_flat = dinv.reshape(NP)
    dinv_col = dinv.reshape(NP, 1)

    h1 = _gat_layer(x1, g1_W1, g1_as1, g1_ad1, g1_b1, src, dst, True)
    emb1 = _gat_layer(h1, g1_W2, g1_as2, g1_ad2, g1_b2, src, dst, False)
    h2 = _gat_layer(x2, g2_W1, g2_as1, g2_ad1, g2_b1, src, dst, True)
    emb2 = _gat_layer(h2, g2_W2, g2_as2, g2_ad2, g2_b2, src, dst, False)

    hc1 = _gcn_layer(x1, c_W1, c_b1, src, dst, dinv_flat, dinv_col, True)
    com1 = _gcn_layer(hc1, c_W2, c_b2, src, dst, dinv_flat, dinv_col, False)
    hc2 = _gcn_layer(x2, c_W1, c_b1, src, dst, dinv_flat, dinv_col, True)
    com2 = _gcn_layer(hc2, c_W2, c_b2, src, dst, dinv_flat, dinv_col, False)

    out = _fusion(emb1, emb2, com1, com2, a_W1, a_b1.reshape(1, 16), a_W2)
    return out[:N]
